# baseline (speedup 1.0000x reference)
import jax
import jax.numpy as jnp
from jax.experimental import pallas as pl
from jax.experimental.pallas import tpu as pltpu

CHARGE_DIM = 64


def kernel(sequence, charge, length):
    B, L = sequence.shape
    D = CHARGE_DIM
    ROW = L * D
    SUB = ROW // 128

    def body(charge_ref, length_ref, out_ref):
        b = pl.program_id(0)
        ch = charge_ref[b]
        cut = length_ref[b] * D
        e = (
            jax.lax.broadcasted_iota(jnp.int32, (SUB, 128), 0) * 128
            + jax.lax.broadcasted_iota(jnp.int32, (SUB, 128), 1)
        )
        val = jnp.where(e < cut, ch, jnp.float32(0.0))
        out_ref[...] = val.reshape(ROW)

    flat = pl.pallas_call(
        body,
        grid=(B,),
        in_specs=[
            pl.BlockSpec(memory_space=pltpu.SMEM),
            pl.BlockSpec(memory_space=pltpu.SMEM),
        ],
        out_specs=pl.BlockSpec((ROW,), lambda b: (b,)),
        out_shape=jax.ShapeDtypeStruct((B * ROW,), jnp.float32),
    )(charge, length)
    return flat.reshape(B, L, D)


# speedup vs baseline: 6.7569x; 6.7569x over previous
import jax
import jax.numpy as jnp
from jax.experimental import pallas as pl
from jax.experimental.pallas import tpu as pltpu

CHARGE_DIM = 64


def kernel(sequence, charge, length):
    B, L = sequence.shape
    D = CHARGE_DIM

    def body(charge_ref, length_ref, out_ref):
        b = pl.program_id(0)
        ch = charge_ref[b]
        ln = length_ref[b]
        pos = jax.lax.broadcasted_iota(jnp.int32, (D, L), 1)
        out_ref[0] = jnp.where(pos < ln, ch, jnp.float32(0.0))

    out_bdl = pl.pallas_call(
        body,
        grid=(B,),
        in_specs=[
            pl.BlockSpec(memory_space=pltpu.SMEM),
            pl.BlockSpec(memory_space=pltpu.SMEM),
        ],
        out_specs=pl.BlockSpec((1, D, L), lambda b: (b, 0, 0)),
        out_shape=jax.ShapeDtypeStruct((B, D, L), jnp.float32),
    )(charge, length)
    return out_bdl.transpose(0, 2, 1)
